# baseline (speedup 1.0000x reference)
import jax
import jax.numpy as jnp
from jax import lax
from jax.experimental import pallas as pl
from jax.experimental.pallas import tpu as pltpu
from jax.experimental.pallas import tpu_sc as plsc

EMBED_DIM = 128
NUM_CORES = 2
NUM_SUBCORES = 16
NUM_WORKERS = NUM_CORES * NUM_SUBCORES
ROWS_PER_GATHER = 8
NUM_CHUNKS = 2


def _sc_gather_chunk(table, idx, chunk, chunk_batch, seq):
    rows_per_worker = chunk_batch // NUM_WORKERS
    idx_per_worker = rows_per_worker * seq
    gw = ROWS_PER_GATHER * seq
    n_gathers = rows_per_worker // ROWS_PER_GATHER
    chunk_off = chunk * chunk_batch * seq

    mesh = plsc.VectorSubcoreMesh(core_axis_name="core", subcore_axis_name="subcore")

    @pl.kernel(
        out_type=jax.ShapeDtypeStruct((chunk_batch, seq, EMBED_DIM), table.dtype),
        mesh=mesh,
        scratch_types=[
            pltpu.VMEM((idx_per_worker,), jnp.int32),
            pltpu.VMEM((gw, EMBED_DIM), jnp.float32),
            pltpu.VMEM((gw, EMBED_DIM), jnp.float32),
            pltpu.SemaphoreType.DMA,
            pltpu.SemaphoreType.DMA,
            pltpu.SemaphoreType.DMA,
        ],
    )
    def gather_kernel(table_hbm, idx_hbm, out_hbm, idx_v, buf0, buf1, gsem, osem0, osem1):
        wid = lax.axis_index("subcore") * NUM_CORES + lax.axis_index("core")
        row_base = wid * rows_per_worker

        pltpu.sync_copy(
            idx_hbm.at[0, pl.ds(chunk_off + row_base * seq, idx_per_worker)], idx_v
        )

        bufs = (buf0, buf1)
        osems = (osem0, osem1)

        def start_gather(g):
            return pltpu.async_copy(
                table_hbm.at[idx_v.at[pl.ds(g * gw, gw)]], bufs[g % 2], gsem
            )

        out_handles = [[], []]
        gather_handle = start_gather(0)
        for g in range(n_gathers):
            gather_handle.wait()
            if g + 1 < n_gathers:
                nxt = (g + 1) % 2
                for h in out_handles[nxt]:
                    h.wait()
                out_handles[nxt] = []
                gather_handle = start_gather(g + 1)
            buf = bufs[g % 2]
            for j in range(ROWS_PER_GATHER):
                out_handles[g % 2].append(
                    pltpu.async_copy(
                        buf.at[pl.ds(j * seq, seq)],
                        out_hbm.at[row_base + g * ROWS_PER_GATHER + j],
                        osems[g % 2],
                    )
                )
        for side in out_handles:
            for h in side:
                h.wait()

    return gather_kernel(table, idx)


def kernel(x, table):
    batch, seq = x.shape
    idx = x.reshape(1, batch * seq).astype(jnp.int32)
    chunk_batch = batch // NUM_CHUNKS

    parts = [
        _sc_gather_chunk(table, idx, c, chunk_batch, seq) for c in range(NUM_CHUNKS)
    ]
    return jnp.concatenate(parts, axis=0) if len(parts) > 1 else parts[0]


# speedup vs baseline: 1.6348x; 1.6348x over previous
import jax
import jax.numpy as jnp
from jax import lax
from jax.experimental import pallas as pl
from jax.experimental.pallas import tpu as pltpu
from jax.experimental.pallas import tpu_sc as plsc

EMBED_DIM = 128
NUM_CORES = 2
NUM_SUBCORES = 16
NUM_WORKERS = NUM_CORES * NUM_SUBCORES
ROWS_PER_GATHER = 4
NBUF = 4


def kernel(x, table):
    batch, seq = x.shape
    num_idx = batch * seq
    idx = x.reshape(1, num_idx).astype(jnp.int32)

    rows_per_worker = batch // NUM_WORKERS
    idx_per_worker = rows_per_worker * seq
    gw = ROWS_PER_GATHER * seq
    n_gathers = rows_per_worker // ROWS_PER_GATHER

    mesh = plsc.VectorSubcoreMesh(core_axis_name="core", subcore_axis_name="subcore")

    @pl.kernel(
        out_type=jax.ShapeDtypeStruct((batch, seq, EMBED_DIM), table.dtype),
        mesh=mesh,
        scratch_types=[
            pltpu.VMEM((idx_per_worker,), jnp.int32),
            pltpu.VMEM((gw, EMBED_DIM), jnp.float32),
            pltpu.VMEM((gw, EMBED_DIM), jnp.float32),
            pltpu.VMEM((gw, EMBED_DIM), jnp.float32),
            pltpu.VMEM((gw, EMBED_DIM), jnp.float32),
            pltpu.SemaphoreType.DMA,
            pltpu.SemaphoreType.DMA,
            pltpu.SemaphoreType.DMA,
            pltpu.SemaphoreType.DMA,
        ],
    )
    def gather_kernel(
        table_hbm, idx_hbm, out_hbm,
        idx_v, buf0, buf1, buf2, buf3, gsem0, gsem1, osem0, osem1,
    ):
        wid = lax.axis_index("subcore") * NUM_CORES + lax.axis_index("core")
        row_base = wid * rows_per_worker

        pltpu.sync_copy(idx_hbm.at[0, pl.ds(row_base * seq, idx_per_worker)], idx_v)

        bufs = (buf0, buf1, buf2, buf3)
        gsems = (gsem0, gsem1)
        osems = (osem0, osem1)

        def start_gather(g):
            return pltpu.async_copy(
                table_hbm.at[idx_v.at[pl.ds(g * gw, gw)]], bufs[g % NBUF], gsems[g % 2]
            )

        gather_handles = [start_gather(0), start_gather(1)]
        out_handles = [[] for _ in range(NBUF)]
        for g in range(n_gathers):
            gather_handles[g % 2].wait()
            if g + 2 < n_gathers:
                nxt = (g + 2) % NBUF
                for h in out_handles[nxt]:
                    h.wait()
                out_handles[nxt] = []
                gather_handles[g % 2] = start_gather(g + 2)
            buf = bufs[g % NBUF]
            for j in range(ROWS_PER_GATHER):
                out_handles[g % NBUF].append(
                    pltpu.async_copy(
                        buf.at[pl.ds(j * seq, seq)],
                        out_hbm.at[row_base + g * ROWS_PER_GATHER + j],
                        osems[g % 2],
                    )
                )
        for side in out_handles:
            for h in side:
                h.wait()

    return gather_kernel(table, idx)


# speedup vs baseline: 1.6431x; 1.0051x over previous
import jax
import jax.numpy as jnp
from jax import lax
from jax.experimental import pallas as pl
from jax.experimental.pallas import tpu as pltpu
from jax.experimental.pallas import tpu_sc as plsc

EMBED_DIM = 128
NUM_CORES = 2
NUM_SUBCORES = 16
NUM_WORKERS = NUM_CORES * NUM_SUBCORES
ROWS_PER_GATHER = 4
NBUF = 4


def kernel(x, table):
    batch, seq = x.shape
    num_idx = batch * seq
    idx = x.reshape(1, num_idx).astype(jnp.int32)

    rows_per_worker = batch // NUM_WORKERS
    idx_per_worker = rows_per_worker * seq
    gw = ROWS_PER_GATHER * seq
    n_gathers = rows_per_worker // ROWS_PER_GATHER

    mesh = plsc.VectorSubcoreMesh(core_axis_name="core", subcore_axis_name="subcore")

    @pl.kernel(
        out_type=jax.ShapeDtypeStruct((batch, seq, EMBED_DIM), table.dtype),
        mesh=mesh,
        scratch_types=[
            pltpu.VMEM((idx_per_worker,), jnp.int32),
            pltpu.VMEM((gw, EMBED_DIM), jnp.float32),
            pltpu.VMEM((gw, EMBED_DIM), jnp.float32),
            pltpu.VMEM((gw, EMBED_DIM), jnp.float32),
            pltpu.VMEM((gw, EMBED_DIM), jnp.float32),
            pltpu.SemaphoreType.DMA,
            pltpu.SemaphoreType.DMA,
            pltpu.SemaphoreType.DMA,
            pltpu.SemaphoreType.DMA,
            pltpu.SemaphoreType.DMA,
        ],
    )
    def gather_kernel(
        table_hbm, idx_hbm, out_hbm,
        idx_v, buf0, buf1, buf2, buf3, gsem0, gsem1, gsem2, osem0, osem1,
    ):
        wid = lax.axis_index("subcore") * NUM_CORES + lax.axis_index("core")
        row_base = wid * rows_per_worker

        pltpu.sync_copy(idx_hbm.at[0, pl.ds(row_base * seq, idx_per_worker)], idx_v)

        bufs = (buf0, buf1, buf2, buf3)
        gsems = (gsem0, gsem1, gsem2)
        osems = (osem0, osem1)

        def start_gather(g):
            return pltpu.async_copy(
                table_hbm.at[idx_v.at[pl.ds(g * gw, gw)]], bufs[g % NBUF], gsems[g % 3]
            )

        gather_handles = [start_gather(0), start_gather(1), start_gather(2)]
        out_handles = [[] for _ in range(NBUF)]
        for g in range(n_gathers):
            gather_handles[g % 3].wait()
            if g + 3 < n_gathers:
                nxt = (g + 3) % NBUF
                for h in out_handles[nxt]:
                    h.wait()
                out_handles[nxt] = []
                gather_handles[g % 3] = start_gather(g + 3)
            buf = bufs[g % NBUF]
            for j in range(ROWS_PER_GATHER):
                out_handles[g % NBUF].append(
                    pltpu.async_copy(
                        buf.at[pl.ds(j * seq, seq)],
                        out_hbm.at[row_base + g * ROWS_PER_GATHER + j],
                        osems[g % 2],
                    )
                )
        for side in out_handles:
            for h in side:
                h.wait()

    return gather_kernel(table, idx)
